# Initial kernel scaffold; baseline (speedup 1.0000x reference)
import jax
import jax.numpy as jnp
from jax.experimental import pallas as pl


def kernel(x, Wg, bg, W1, b1, W2, b2, gamma, beta):
    raise NotImplementedError("write your pallas kernel here")


# speedup vs baseline: 1.0503x; 1.0503x over previous
import functools

import jax
import jax.numpy as jnp
from jax.experimental import pallas as pl
from jax.experimental.pallas import tpu as pltpu

B, N, D = 2, 4096, 1024
S, K, H = 64, 2, 512

CN = 512
NB = N // CN


def _route_kernel(x_ref, wg_ref, bg_ref, comp_ref, cnt_ref):
    nb = pl.program_id(1)
    xc = x_ref[0]
    logits = jax.lax.dot_general(
        xc, wg_ref[...], (((1,), (0,)), ((), ())),
        preferred_element_type=jnp.float32) + bg_ref[...]
    m = jnp.max(logits, axis=-1, keepdims=True)
    e = jnp.exp(logits - m)
    probs = e / jnp.sum(e, axis=-1, keepdims=True)
    iota = jax.lax.broadcasted_iota(jnp.int32, probs.shape, 1)
    m1 = jnp.max(probs, axis=-1, keepdims=True)
    i1 = jnp.min(jnp.where(probs == m1, iota, S), axis=-1, keepdims=True)
    mask1 = iota == i1
    p2 = jnp.where(mask1, -jnp.inf, probs)
    m2 = jnp.max(p2, axis=-1, keepdims=True)
    i2 = jnp.min(jnp.where(p2 == m2, iota, S), axis=-1, keepdims=True)
    w = jnp.where(mask1 | (iota == i2), probs, 0.0)

    part = jax.lax.dot_general(
        w, xc, (((0,), (0,)), ((), ())),
        preferred_element_type=jnp.float32)
    csum = jnp.sum(w, axis=0)[None, :]

    @pl.when(nb == 0)
    def _init():
        comp_ref[0] = part
        cnt_ref[...] = csum

    @pl.when(nb > 0)
    def _acc():
        comp_ref[0] += part
        cnt_ref[...] += csum

    @pl.when(nb == NB - 1)
    def _final():
        comp_ref[0] = comp_ref[0] / (cnt_ref[0][:, None] + 1e-9)


def _mlp_kernel(comp_ref, w1_ref, b1_ref, w2_ref, b2_ref, g_ref, be_ref,
                out_ref):
    c = comp_ref[0]
    h = jax.lax.dot_general(
        c, w1_ref[0], (((1,), (0,)), ((), ())),
        preferred_element_type=jnp.float32) + b1_ref[0]
    h = 0.5 * h * (1.0 + jax.lax.erf(h * 0.7071067811865476))
    y = jax.lax.dot_general(
        h, w2_ref[0], (((1,), (0,)), ((), ())),
        preferred_element_type=jnp.float32) + b2_ref[0]
    mu = jnp.mean(y, axis=-1, keepdims=True)
    var = jnp.mean((y - mu) ** 2, axis=-1, keepdims=True)
    ln = (y - mu) * jax.lax.rsqrt(var + 1e-5) * g_ref[0] + be_ref[0]
    out_ref[0] = c + ln


@functools.partial(jax.jit)
def kernel(x, Wg, bg, W1, b1, W2, b2, gamma, beta):
    comp = pl.pallas_call(
        _route_kernel,
        grid=(B, NB),
        in_specs=[
            pl.BlockSpec((1, CN, D), lambda b, n: (b, n, 0)),
            pl.BlockSpec((D, S), lambda b, n: (0, 0)),
            pl.BlockSpec((S,), lambda b, n: (0,)),
        ],
        out_specs=pl.BlockSpec((1, S, D), lambda b, n: (b, 0, 0)),
        out_shape=jax.ShapeDtypeStruct((B, S, D), jnp.float32),
        scratch_shapes=[pltpu.VMEM((1, S), jnp.float32)],
    )(x, Wg, bg)

    compT = comp.transpose(1, 0, 2)
    final = pl.pallas_call(
        _mlp_kernel,
        grid=(S,),
        in_specs=[
            pl.BlockSpec((1, B, D), lambda s: (s, 0, 0)),
            pl.BlockSpec((1, D, H), lambda s: (s, 0, 0)),
            pl.BlockSpec((1, 1, H), lambda s: (s, 0, 0)),
            pl.BlockSpec((1, H, D), lambda s: (s, 0, 0)),
            pl.BlockSpec((1, 1, D), lambda s: (s, 0, 0)),
            pl.BlockSpec((1, 1, D), lambda s: (s, 0, 0)),
            pl.BlockSpec((1, 1, D), lambda s: (s, 0, 0)),
        ],
        out_specs=pl.BlockSpec((1, B, D), lambda s: (s, 0, 0)),
        out_shape=jax.ShapeDtypeStruct((S, B, D), jnp.float32),
    )(compT, W1, b1[:, None, :], W2, b2[:, None, :],
      gamma[:, None, :], beta[:, None, :]).transpose(1, 0, 2)

    aux_loss = jnp.array(0.0, dtype=jnp.float32)
    return (final, aux_loss)


# speedup vs baseline: 1.1168x; 1.0632x over previous
import functools

import jax
import jax.numpy as jnp
from jax.experimental import pallas as pl
from jax.experimental.pallas import tpu as pltpu

B, N, D = 2, 4096, 1024
S, K, H = 64, 2, 512

CN = 1024
NB = N // CN


def _route_kernel(x_ref, wg_ref, bg_ref, comp_ref, cnt_ref):
    nb = pl.program_id(1)
    xc = x_ref[0]
    logits = jax.lax.dot_general(
        xc, wg_ref[...], (((1,), (0,)), ((), ())),
        preferred_element_type=jnp.float32) + bg_ref[...]
    m1 = jnp.max(logits, axis=-1, keepdims=True)
    l2 = jnp.where(logits >= m1, -jnp.inf, logits)
    m2 = jnp.max(l2, axis=-1, keepdims=True)
    e = jnp.exp(logits - m1)
    sum_e = jnp.sum(e, axis=-1, keepdims=True)
    w = jnp.where(logits >= m2, e, 0.0) / sum_e

    part = jax.lax.dot_general(
        w, xc, (((0,), (0,)), ((), ())),
        preferred_element_type=jnp.float32)
    csum = jnp.sum(w, axis=0)[None, :]

    @pl.when(nb == 0)
    def _init():
        comp_ref[0] = part
        cnt_ref[...] = csum

    @pl.when(nb > 0)
    def _acc():
        comp_ref[0] += part
        cnt_ref[...] += csum

    @pl.when(nb == NB - 1)
    def _final():
        comp_ref[0] = comp_ref[0] / (cnt_ref[0][:, None] + 1e-9)


def _mlp_kernel(comp_ref, w1_ref, b1_ref, w2_ref, b2_ref, g_ref, be_ref,
                out_ref):
    c = comp_ref[0]
    h = jax.lax.dot_general(
        c, w1_ref[0], (((1,), (0,)), ((), ())),
        preferred_element_type=jnp.float32) + b1_ref[0]
    h = 0.5 * h * (1.0 + jax.lax.erf(h * 0.7071067811865476))
    y = jax.lax.dot_general(
        h, w2_ref[0], (((1,), (0,)), ((), ())),
        preferred_element_type=jnp.float32) + b2_ref[0]
    mu = jnp.mean(y, axis=-1, keepdims=True)
    var = jnp.mean((y - mu) ** 2, axis=-1, keepdims=True)
    ln = (y - mu) * jax.lax.rsqrt(var + 1e-5) * g_ref[0] + be_ref[0]
    out_ref[0] = c + ln


@functools.partial(jax.jit)
def kernel(x, Wg, bg, W1, b1, W2, b2, gamma, beta):
    comp = pl.pallas_call(
        _route_kernel,
        grid=(B, NB),
        in_specs=[
            pl.BlockSpec((1, CN, D), lambda b, n: (b, n, 0)),
            pl.BlockSpec((D, S), lambda b, n: (0, 0)),
            pl.BlockSpec((S,), lambda b, n: (0,)),
        ],
        out_specs=pl.BlockSpec((1, S, D), lambda b, n: (b, 0, 0)),
        out_shape=jax.ShapeDtypeStruct((B, S, D), jnp.float32),
        scratch_shapes=[pltpu.VMEM((1, S), jnp.float32)],
        compiler_params=pltpu.CompilerParams(
            dimension_semantics=("parallel", "arbitrary")),
    )(x, Wg, bg)

    compT = comp.transpose(1, 0, 2)
    final = pl.pallas_call(
        _mlp_kernel,
        grid=(S,),
        in_specs=[
            pl.BlockSpec((1, B, D), lambda s: (s, 0, 0)),
            pl.BlockSpec((1, D, H), lambda s: (s, 0, 0)),
            pl.BlockSpec((1, 1, H), lambda s: (s, 0, 0)),
            pl.BlockSpec((1, H, D), lambda s: (s, 0, 0)),
            pl.BlockSpec((1, 1, D), lambda s: (s, 0, 0)),
            pl.BlockSpec((1, 1, D), lambda s: (s, 0, 0)),
            pl.BlockSpec((1, 1, D), lambda s: (s, 0, 0)),
        ],
        out_specs=pl.BlockSpec((1, B, D), lambda s: (s, 0, 0)),
        out_shape=jax.ShapeDtypeStruct((S, B, D), jnp.float32),
        compiler_params=pltpu.CompilerParams(
            dimension_semantics=("parallel",)),
    )(compT, W1, b1[:, None, :], W2, b2[:, None, :],
      gamma[:, None, :], beta[:, None, :]).transpose(1, 0, 2)

    aux_loss = jnp.array(0.0, dtype=jnp.float32)
    return (final, aux_loss)


# speedup vs baseline: 1.3112x; 1.1741x over previous
import functools

import jax
import jax.numpy as jnp
from jax.experimental import pallas as pl
from jax.experimental.pallas import tpu as pltpu

B, N, D = 2, 4096, 1024
S, K, H = 64, 2, 512

CN = 1024
NB = N // CN


def _route_kernel(x_ref, wg_ref, bg_ref, comp_ref, cnt_ref):
    nb = pl.program_id(1)
    xc = x_ref[0]
    logits = jax.lax.dot_general(
        xc, wg_ref[...], (((1,), (0,)), ((), ())),
        preferred_element_type=jnp.float32) + bg_ref[...]
    m1 = jnp.max(logits, axis=-1, keepdims=True)
    l2 = jnp.where(logits >= m1, -jnp.inf, logits)
    m2 = jnp.max(l2, axis=-1, keepdims=True)
    e = jnp.exp(logits - m1)
    sum_e = jnp.sum(e, axis=-1, keepdims=True)
    w = jnp.where(logits >= m2, e, 0.0) / sum_e

    part = jax.lax.dot_general(
        w, xc, (((0,), (0,)), ((), ())),
        preferred_element_type=jnp.float32)
    csum = jnp.sum(w, axis=0)[None, :]

    @pl.when(nb == 0)
    def _init():
        comp_ref[0] = part
        cnt_ref[...] = csum

    @pl.when(nb > 0)
    def _acc():
        comp_ref[0] += part
        cnt_ref[...] += csum

    @pl.when(nb == NB - 1)
    def _final():
        comp_ref[0] = comp_ref[0] / (cnt_ref[0][:, None] + 1e-9)


ST = 2


def _mlp_kernel(comp_ref, w1_ref, b1_ref, w2_ref, b2_ref, g_ref, be_ref,
                out_ref):
    for j in range(ST):
        c = comp_ref[j]
        h = jax.lax.dot_general(
            c, w1_ref[j], (((1,), (0,)), ((), ())),
            preferred_element_type=jnp.float32) + b1_ref[j]
        h = 0.5 * h * (1.0 + jax.lax.erf(h * 0.7071067811865476))
        y = jax.lax.dot_general(
            h, w2_ref[j], (((1,), (0,)), ((), ())),
            preferred_element_type=jnp.float32) + b2_ref[j]
        mu = jnp.mean(y, axis=-1, keepdims=True)
        var = jnp.mean((y - mu) ** 2, axis=-1, keepdims=True)
        ln = (y - mu) * jax.lax.rsqrt(var + 1e-5) * g_ref[j] + be_ref[j]
        out_ref[j] = c + ln


@functools.partial(jax.jit)
def kernel(x, Wg, bg, W1, b1, W2, b2, gamma, beta):
    comp = pl.pallas_call(
        _route_kernel,
        grid=(B, NB),
        in_specs=[
            pl.BlockSpec((1, CN, D), lambda b, n: (b, n, 0)),
            pl.BlockSpec((D, S), lambda b, n: (0, 0)),
            pl.BlockSpec((S,), lambda b, n: (0,)),
        ],
        out_specs=pl.BlockSpec((1, S, D), lambda b, n: (b, 0, 0)),
        out_shape=jax.ShapeDtypeStruct((B, S, D), jnp.float32),
        scratch_shapes=[pltpu.VMEM((1, S), jnp.float32)],
        compiler_params=pltpu.CompilerParams(
            dimension_semantics=("parallel", "arbitrary")),
    )(x, Wg, bg)

    compT = comp.transpose(1, 0, 2)
    final = pl.pallas_call(
        _mlp_kernel,
        grid=(S // ST,),
        in_specs=[
            pl.BlockSpec((ST, B, D), lambda s: (s, 0, 0)),
            pl.BlockSpec((ST, D, H), lambda s: (s, 0, 0)),
            pl.BlockSpec((ST, 1, H), lambda s: (s, 0, 0)),
            pl.BlockSpec((ST, H, D), lambda s: (s, 0, 0)),
            pl.BlockSpec((ST, 1, D), lambda s: (s, 0, 0)),
            pl.BlockSpec((ST, 1, D), lambda s: (s, 0, 0)),
            pl.BlockSpec((ST, 1, D), lambda s: (s, 0, 0)),
        ],
        out_specs=pl.BlockSpec((ST, B, D), lambda s: (s, 0, 0)),
        out_shape=jax.ShapeDtypeStruct((S, B, D), jnp.float32),
        compiler_params=pltpu.CompilerParams(
            dimension_semantics=("parallel",)),
    )(compT, W1, b1[:, None, :], W2, b2[:, None, :],
      gamma[:, None, :], beta[:, None, :]).transpose(1, 0, 2)

    aux_loss = jnp.array(0.0, dtype=jnp.float32)
    return (final, aux_loss)
